# Initial kernel scaffold; baseline (speedup 1.0000x reference)
import jax
import jax.numpy as jnp
from jax.experimental import pallas as pl


def kernel(hidden_states, top_k_index, top_k_weights, gate_up_proj, down_proj):
    raise NotImplementedError("write your pallas kernel here")


# speedup vs baseline: 1.0683x; 1.0683x over previous
import functools

import jax
import jax.numpy as jnp
from jax.experimental import pallas as pl
from jax.experimental.pallas import tpu as pltpu

NUM_EXPERTS = 8
TOP_K = 2
HIDDEN = 2048
INTER = 1024
TOKENS = 4096

_M = 512


def _dense_body(idx_ref, w_ref, x_ref, gup_ref, down_ref, out_ref):
    e = pl.program_id(1)
    gu = jnp.dot(x_ref[...], gup_ref[0], preferred_element_type=jnp.float32)
    gate = gu[:, :INTER]
    up = gu[:, INTER:]
    act = (gate * jax.nn.sigmoid(gate) * up).astype(jnp.bfloat16)
    cur = jnp.dot(act, down_ref[0], preferred_element_type=jnp.float32)
    wv = jnp.sum(jnp.where(idx_ref[...] == e, w_ref[...], 0.0), axis=1)
    contrib = cur * wv[:, None]

    @pl.when(e == 0)
    def _():
        out_ref[...] = contrib

    @pl.when(e > 0)
    def _():
        out_ref[...] += contrib


def kernel(hidden_states, top_k_index, top_k_weights, gate_up_proj, down_proj):
    x = hidden_states.astype(jnp.bfloat16)
    idx = top_k_index.astype(jnp.int32)
    gup_t = jnp.swapaxes(gate_up_proj, 1, 2).astype(jnp.bfloat16)
    down_t = jnp.swapaxes(down_proj, 1, 2).astype(jnp.bfloat16)

    grid = (TOKENS // _M, NUM_EXPERTS)
    out = pl.pallas_call(
        _dense_body,
        grid=grid,
        in_specs=[
            pl.BlockSpec((_M, TOP_K), lambda tb, e: (tb, 0)),
            pl.BlockSpec((_M, TOP_K), lambda tb, e: (tb, 0)),
            pl.BlockSpec((_M, HIDDEN), lambda tb, e: (tb, 0)),
            pl.BlockSpec((1, HIDDEN, 2 * INTER), lambda tb, e: (e, 0, 0)),
            pl.BlockSpec((1, INTER, HIDDEN), lambda tb, e: (e, 0, 0)),
        ],
        out_specs=pl.BlockSpec((_M, HIDDEN), lambda tb, e: (tb, 0)),
        out_shape=jax.ShapeDtypeStruct((TOKENS, HIDDEN), jnp.float32),
    )(idx, top_k_weights, x, gup_t, down_t)
    return out
